# baseline (speedup 1.0000x reference)
import functools
import math

import jax
import jax.numpy as jnp
from jax.experimental import pallas as pl
from jax.experimental.pallas import tpu as pltpu


def _gcn_body(adj_ref, xk_ref, xm_ref, h0_ref, w1_ref, w2_ref, out_ref,
              acc_ref, *, nk: int, nb: int):
    k = pl.program_id(1)

    @pl.when(k == 0)
    def _():
        acc_ref[...] = jnp.zeros_like(acc_ref)

    a = adj_ref[...]
    for b in range(nb):
        acc_ref[b] += jnp.dot(a, xk_ref[b],
                              preferred_element_type=jnp.float32)

    @pl.when(k == nk - 1)
    def _():
        for b in range(nb):
            out_ref[b] = (
                jnp.dot(acc_ref[b], w1_ref[...],
                        preferred_element_type=jnp.float32)
                + jnp.dot(h0_ref[b], w2_ref[...],
                          preferred_element_type=jnp.float32)
                + xm_ref[b]
            )


def kernel(prott5_emd, adj, h0, weight, lamda, alpha, l):
    B, N, d = prott5_emd.shape
    theta = jnp.minimum(1.0, jnp.log(lamda / l + 1.0)).astype(jnp.float32)
    alpha = jnp.asarray(alpha, jnp.float32)
    eye = jnp.eye(d, dtype=jnp.float32)
    w1a = theta * weight[:d] + (1.0 - theta) * (1.0 - alpha) * eye
    w2a = theta * weight[d:] + (1.0 - theta) * alpha * eye

    bm, bk = 512, 512
    nm, nk = N // bm, N // bk

    out = pl.pallas_call(
        functools.partial(_gcn_body, nk=nk, nb=B),
        grid=(nm, nk),
        in_specs=[
            pl.BlockSpec((bm, bk), lambda m, k: (m, k)),
            pl.BlockSpec((B, bk, d), lambda m, k: (0, k, 0)),
            pl.BlockSpec((B, bm, d), lambda m, k: (0, m, 0)),
            pl.BlockSpec((B, bm, d), lambda m, k: (0, m, 0)),
            pl.BlockSpec((d, d), lambda m, k: (0, 0)),
            pl.BlockSpec((d, d), lambda m, k: (0, 0)),
        ],
        out_specs=pl.BlockSpec((B, bm, d), lambda m, k: (0, m, 0)),
        out_shape=jax.ShapeDtypeStruct((B, N, d), jnp.float32),
        scratch_shapes=[pltpu.VMEM((B, bm, d), jnp.float32)],
        compiler_params=pltpu.CompilerParams(
            dimension_semantics=("arbitrary", "arbitrary"),
        ),
    )(adj, prott5_emd, prott5_emd, h0, w1a, w2a)
    return out


# speedup vs baseline: 1.0891x; 1.0891x over previous
import functools

import jax
import jax.numpy as jnp
from jax.experimental import pallas as pl
from jax.experimental.pallas import tpu as pltpu


def _gcn_body(adj_ref, x_ref, h0_ref, w1_ref, w2_ref, out_ref,
              acc_ref, *, nk: int, bk: int, nb: int):
    k = pl.program_id(1)

    @pl.when(k == 0)
    def _():
        acc_ref[...] = jnp.zeros_like(acc_ref)

    a = adj_ref[...].astype(jnp.bfloat16)
    for b in range(nb):
        xk = x_ref[b, pl.ds(k * bk, bk), :].astype(jnp.bfloat16)
        acc_ref[b] += jnp.dot(a, xk, preferred_element_type=jnp.float32)

    @pl.when(k == nk - 1)
    def _():
        m = pl.program_id(0)
        bm = out_ref.shape[1]
        for b in range(nb):
            out_ref[b] = (
                jnp.dot(acc_ref[b], w1_ref[...],
                        preferred_element_type=jnp.float32)
                + jnp.dot(h0_ref[b], w2_ref[...],
                          preferred_element_type=jnp.float32)
                + x_ref[b, pl.ds(m * bm, bm), :]
            )


def kernel(prott5_emd, adj, h0, weight, lamda, alpha, l):
    B, N, d = prott5_emd.shape
    theta = jnp.minimum(1.0, jnp.log(lamda / l + 1.0)).astype(jnp.float32)
    alpha = jnp.asarray(alpha, jnp.float32)
    eye = jnp.eye(d, dtype=jnp.float32)
    w1a = theta * weight[:d] + (1.0 - theta) * (1.0 - alpha) * eye
    w2a = theta * weight[d:] + (1.0 - theta) * alpha * eye

    bm, bk = 512, 512
    nm, nk = N // bm, N // bk

    out = pl.pallas_call(
        functools.partial(_gcn_body, nk=nk, bk=bk, nb=B),
        grid=(nm, nk),
        in_specs=[
            pl.BlockSpec((bm, bk), lambda m, k: (m, k)),
            pl.BlockSpec((B, N, d), lambda m, k: (0, 0, 0)),
            pl.BlockSpec((B, bm, d), lambda m, k: (0, m, 0)),
            pl.BlockSpec((d, d), lambda m, k: (0, 0)),
            pl.BlockSpec((d, d), lambda m, k: (0, 0)),
        ],
        out_specs=pl.BlockSpec((B, bm, d), lambda m, k: (0, m, 0)),
        out_shape=jax.ShapeDtypeStruct((B, N, d), jnp.float32),
        scratch_shapes=[pltpu.VMEM((B, bm, d), jnp.float32)],
        compiler_params=pltpu.CompilerParams(
            dimension_semantics=("arbitrary", "arbitrary"),
        ),
    )(adj, prott5_emd, h0, w1a, w2a)
    return out


# speedup vs baseline: 1.4320x; 1.3148x over previous
import functools

import jax
import jax.numpy as jnp
from jax.experimental import pallas as pl
from jax.experimental.pallas import tpu as pltpu


def _gcn_body(adj_ref, xb_ref, xm_ref, h0_ref, w1_ref, w2_ref, out_ref,
              *, nb: int):
    a = adj_ref[...].astype(jnp.bfloat16)
    for b in range(nb):
        hi = jnp.dot(a, xb_ref[b], preferred_element_type=jnp.float32)
        out_ref[b] = (
            jnp.dot(hi, w1_ref[...], preferred_element_type=jnp.float32)
            + jnp.dot(h0_ref[b], w2_ref[...],
                      preferred_element_type=jnp.float32)
            + xm_ref[b]
        )


def kernel(prott5_emd, adj, h0, weight, lamda, alpha, l):
    B, N, d = prott5_emd.shape
    theta = jnp.minimum(1.0, jnp.log(lamda / l + 1.0)).astype(jnp.float32)
    alpha = jnp.asarray(alpha, jnp.float32)
    eye = jnp.eye(d, dtype=jnp.float32)
    w1a = theta * weight[:d] + (1.0 - theta) * (1.0 - alpha) * eye
    w2a = theta * weight[d:] + (1.0 - theta) * alpha * eye

    x_bf16 = prott5_emd.astype(jnp.bfloat16)

    bm = 512
    nm = N // bm

    out = pl.pallas_call(
        functools.partial(_gcn_body, nb=B),
        grid=(nm,),
        in_specs=[
            pl.BlockSpec((bm, N), lambda m: (m, 0)),
            pl.BlockSpec((B, N, d), lambda m: (0, 0, 0)),
            pl.BlockSpec((B, bm, d), lambda m: (0, m, 0)),
            pl.BlockSpec((B, bm, d), lambda m: (0, m, 0)),
            pl.BlockSpec((d, d), lambda m: (0, 0)),
            pl.BlockSpec((d, d), lambda m: (0, 0)),
        ],
        out_specs=pl.BlockSpec((B, bm, d), lambda m: (0, m, 0)),
        out_shape=jax.ShapeDtypeStruct((B, N, d), jnp.float32),
        compiler_params=pltpu.CompilerParams(
            dimension_semantics=("arbitrary",),
        ),
    )(adj, x_bf16, prott5_emd, h0, w1a, w2a)
    return out


# speedup vs baseline: 1.9741x; 1.3786x over previous
import functools

import jax
import jax.numpy as jnp
from jax.experimental import pallas as pl
from jax.experimental.pallas import tpu as pltpu


def _gcn_body(adj_ref, xc_ref, xm_ref, h0_ref, w1_ref, w2_ref, out_ref,
              *, nb: int, d: int):
    a = adj_ref[...].astype(jnp.bfloat16)
    hi2 = jnp.dot(a, xc_ref[...], preferred_element_type=jnp.float32)
    for b in range(nb):
        out_ref[b] = (
            jnp.dot(hi2[:, b * d:(b + 1) * d], w1_ref[...],
                    preferred_element_type=jnp.float32)
            + jnp.dot(h0_ref[b], w2_ref[...],
                      preferred_element_type=jnp.float32)
            + xm_ref[b]
        )


def kernel(prott5_emd, adj, h0, weight, lamda, alpha, l):
    B, N, d = prott5_emd.shape
    theta = jnp.minimum(1.0, jnp.log(lamda / l + 1.0)).astype(jnp.float32)
    alpha = jnp.asarray(alpha, jnp.float32)
    eye = jnp.eye(d, dtype=jnp.float32)
    w1a = theta * weight[:d] + (1.0 - theta) * (1.0 - alpha) * eye
    w2a = theta * weight[d:] + (1.0 - theta) * alpha * eye

    xc = jnp.moveaxis(prott5_emd, 0, 1).reshape(N, B * d).astype(jnp.bfloat16)

    bm = 1024
    nm = N // bm

    out = pl.pallas_call(
        functools.partial(_gcn_body, nb=B, d=d),
        grid=(nm,),
        in_specs=[
            pl.BlockSpec((bm, N), lambda m: (m, 0)),
            pl.BlockSpec((N, B * d), lambda m: (0, 0)),
            pl.BlockSpec((B, bm, d), lambda m: (0, m, 0)),
            pl.BlockSpec((B, bm, d), lambda m: (0, m, 0)),
            pl.BlockSpec((d, d), lambda m: (0, 0)),
            pl.BlockSpec((d, d), lambda m: (0, 0)),
        ],
        out_specs=pl.BlockSpec((B, bm, d), lambda m: (0, m, 0)),
        out_shape=jax.ShapeDtypeStruct((B, N, d), jnp.float32),
        compiler_params=pltpu.CompilerParams(
            dimension_semantics=("arbitrary",),
        ),
    )(adj, xc, prott5_emd, h0, w1a, w2a)
    return out


# speedup vs baseline: 2.2426x; 1.1360x over previous
import functools

import jax
import jax.numpy as jnp
from jax.experimental import pallas as pl
from jax.experimental.pallas import tpu as pltpu


def _gcn_body(adj_ref, x_ref, h0_ref, w1_ref, w2_ref, out_ref, xc_ref,
              *, nb: int, d: int):
    @pl.when(pl.program_id(0) == 0)
    def _():
        for b in range(nb):
            xc_ref[:, b * d:(b + 1) * d] = x_ref[b].astype(jnp.bfloat16)

    a = adj_ref[...].astype(jnp.bfloat16)
    hi2 = jnp.dot(a, xc_ref[...], preferred_element_type=jnp.float32)
    m = pl.program_id(0)
    bm = out_ref.shape[1]
    for b in range(nb):
        out_ref[b] = (
            jnp.dot(hi2[:, b * d:(b + 1) * d], w1_ref[...],
                    preferred_element_type=jnp.float32)
            + jnp.dot(h0_ref[b], w2_ref[...],
                      preferred_element_type=jnp.float32)
            + x_ref[b, pl.ds(m * bm, bm), :]
        )


def kernel(prott5_emd, adj, h0, weight, lamda, alpha, l):
    B, N, d = prott5_emd.shape
    theta = jnp.minimum(1.0, jnp.log(lamda / l + 1.0)).astype(jnp.float32)
    alpha = jnp.asarray(alpha, jnp.float32)
    eye = jnp.eye(d, dtype=jnp.float32)
    w1a = theta * weight[:d] + (1.0 - theta) * (1.0 - alpha) * eye
    w2a = theta * weight[d:] + (1.0 - theta) * alpha * eye

    bm = 1024
    nm = N // bm

    out = pl.pallas_call(
        functools.partial(_gcn_body, nb=B, d=d),
        grid=(nm,),
        in_specs=[
            pl.BlockSpec((bm, N), lambda m: (m, 0)),
            pl.BlockSpec((B, N, d), lambda m: (0, 0, 0)),
            pl.BlockSpec((B, bm, d), lambda m: (0, m, 0)),
            pl.BlockSpec((d, d), lambda m: (0, 0)),
            pl.BlockSpec((d, d), lambda m: (0, 0)),
        ],
        out_specs=pl.BlockSpec((B, bm, d), lambda m: (0, m, 0)),
        out_shape=jax.ShapeDtypeStruct((B, N, d), jnp.float32),
        scratch_shapes=[pltpu.VMEM((N, B * d), jnp.bfloat16)],
        compiler_params=pltpu.CompilerParams(
            dimension_semantics=("arbitrary",),
        ),
    )(adj, prott5_emd, h0, w1a, w2a)
    return out


# speedup vs baseline: 2.3194x; 1.0343x over previous
import functools

import jax
import jax.numpy as jnp
from jax.experimental import pallas as pl
from jax.experimental.pallas import tpu as pltpu


def _gcn_body(adj_ref, x_ref, h0_ref, w1_ref, w2_ref, out_ref, xc_ref,
              *, nb: int, d: int):
    @pl.when(pl.program_id(0) == 0)
    def _():
        for b in range(nb):
            xc_ref[:, b * d:(b + 1) * d] = x_ref[b].astype(jnp.bfloat16)

    a = adj_ref[...].astype(jnp.bfloat16)
    hi2 = jnp.dot(a, xc_ref[...], preferred_element_type=jnp.float32)
    m = pl.program_id(0)
    bm = out_ref.shape[1]
    for b in range(nb):
        out_ref[b] = (
            jnp.dot(hi2[:, b * d:(b + 1) * d], w1_ref[...],
                    preferred_element_type=jnp.float32)
            + jnp.dot(h0_ref[b], w2_ref[...],
                      preferred_element_type=jnp.float32)
            + x_ref[b, pl.ds(m * bm, bm), :]
        )


def kernel(prott5_emd, adj, h0, weight, lamda, alpha, l):
    B, N, d = prott5_emd.shape
    theta = jnp.minimum(1.0, jnp.log(lamda / l + 1.0)).astype(jnp.float32)
    alpha = jnp.asarray(alpha, jnp.float32)
    eye = jnp.eye(d, dtype=jnp.float32)
    w1a = theta * weight[:d] + (1.0 - theta) * (1.0 - alpha) * eye
    w2a = theta * weight[d:] + (1.0 - theta) * alpha * eye

    bm = 512
    nm = N // bm

    out = pl.pallas_call(
        functools.partial(_gcn_body, nb=B, d=d),
        grid=(nm,),
        in_specs=[
            pl.BlockSpec((bm, N), lambda m: (m, 0)),
            pl.BlockSpec((B, N, d), lambda m: (0, 0, 0)),
            pl.BlockSpec((B, bm, d), lambda m: (0, m, 0)),
            pl.BlockSpec((d, d), lambda m: (0, 0)),
            pl.BlockSpec((d, d), lambda m: (0, 0)),
        ],
        out_specs=pl.BlockSpec((B, bm, d), lambda m: (0, m, 0)),
        out_shape=jax.ShapeDtypeStruct((B, N, d), jnp.float32),
        scratch_shapes=[pltpu.VMEM((N, B * d), jnp.bfloat16)],
        compiler_params=pltpu.CompilerParams(
            dimension_semantics=("arbitrary",),
        ),
    )(adj, prott5_emd, h0, w1a, w2a)
    return out
